# Initial kernel scaffold; baseline (speedup 1.0000x reference)
import jax
import jax.numpy as jnp
from jax.experimental import pallas as pl


def kernel(index, rate, starttime, endtime, t):
    raise NotImplementedError("write your pallas kernel here")


# speedup vs baseline: 11.3843x; 11.3843x over previous
import functools

import jax
import jax.numpy as jnp
from jax import lax
from jax.experimental import pallas as pl
from jax.experimental.pallas import tpu as pltpu
from jax.experimental.pallas import tpu_sc as plsc

N = 2_000_000
OUT_SIZE = 1_000_000
LANES = 128
ROWS = N // LANES
CHUNK_ROWS = 16
FULL_CHUNKS = ROWS // CHUNK_ROWS
TAIL_ROWS = ROWS - FULL_CHUNKS * CHUNK_ROWS
NUM_TILES = 32
ACC_SIZE = 1 << 20
STRIPE = ACC_SIZE // 16
ZBUF = 8192


def _sc_body(idx_hbm, rate_hbm, st_hbm, en_hbm, t_hbm, out_hbm,
             t_v, idx_v, val_v, st_v, en_v, zbuf, acc):
    cid = lax.axis_index("c")
    sid = lax.axis_index("s")
    wid = sid * 2 + cid

    def zero_body(i, carry):
        zbuf[pl.ds(i * 16, 16)] = jnp.zeros((16,), jnp.float32)
        return carry
    lax.fori_loop(0, ZBUF // 16, zero_body, 0)
    for z in range(STRIPE // ZBUF):
        pltpu.sync_copy(zbuf, acc.at[pl.ds(sid * STRIPE + z * ZBUF, ZBUF)])

    pltpu.sync_copy(t_hbm, t_v)
    t = t_v[...]

    plsc.subcore_barrier()

    def do_chunk(row0, nrows):
        pltpu.sync_copy(idx_hbm.at[pl.ds(row0, nrows)], idx_v.at[pl.ds(0, nrows)])
        pltpu.sync_copy(rate_hbm.at[pl.ds(row0, nrows)], val_v.at[pl.ds(0, nrows)])
        pltpu.sync_copy(st_hbm.at[pl.ds(row0, nrows)], st_v.at[pl.ds(0, nrows)])
        pltpu.sync_copy(en_hbm.at[pl.ds(row0, nrows)], en_v.at[pl.ds(0, nrows)])
        for j in range(nrows):
            def mask_body(i, carry, j=j):
                c = i * 16
                s = st_v[j, pl.ds(c, 16)]
                e = en_v[j, pl.ds(c, 16)]
                r = val_v[j, pl.ds(c, 16)]
                keep = (s <= t) & (t < e)
                val_v[j, pl.ds(c, 16)] = jnp.where(keep, r, jnp.float32(0.0))
                return carry
            lax.fori_loop(0, LANES // 16, mask_body, 0)
        for j in range(nrows):
            pltpu.sync_copy(val_v.at[j], acc.at[idx_v.at[j]], add=True)

    nchunks = (FULL_CHUNKS - 1 - wid) // NUM_TILES + 1

    def chunk_body(jj, carry):
        row0 = (wid + jj * NUM_TILES) * CHUNK_ROWS
        do_chunk(row0, CHUNK_ROWS)
        return carry
    lax.fori_loop(0, nchunks, chunk_body, 0)

    @pl.when(wid == 16)
    def _tail():
        do_chunk(FULL_CHUNKS * CHUNK_ROWS, TAIL_ROWS)

    plsc.subcore_barrier()

    for z in range(STRIPE // ZBUF):
        off = sid * STRIPE + z * ZBUF
        pltpu.sync_copy(acc.at[pl.ds(off, ZBUF)], zbuf)
        pltpu.sync_copy(zbuf, out_hbm.at[cid, pl.ds(off, ZBUF)])


_sc_call = pl.kernel(
    _sc_body,
    mesh=plsc.VectorSubcoreMesh(core_axis_name="c", subcore_axis_name="s"),
    out_type=jax.ShapeDtypeStruct((2, ACC_SIZE), jnp.float32),
    scratch_types=[
        pltpu.VMEM((16,), jnp.float32),
        pltpu.VMEM((CHUNK_ROWS, LANES), jnp.int32),
        pltpu.VMEM((CHUNK_ROWS, LANES), jnp.float32),
        pltpu.VMEM((CHUNK_ROWS, LANES), jnp.float32),
        pltpu.VMEM((CHUNK_ROWS, LANES), jnp.float32),
        pltpu.VMEM((ZBUF,), jnp.float32),
        pltpu.VMEM_SHARED((ACC_SIZE,), jnp.float32),
    ],
)


def _combine_body(p_ref, o_ref):
    o_ref[...] = p_ref[0] + p_ref[1]


def _combine(partials):
    p3 = partials.reshape(2, 1024, 1024)
    out = pl.pallas_call(
        _combine_body,
        out_shape=jax.ShapeDtypeStruct((1024, 1024), jnp.float32),
    )(p3)
    return out.reshape(ACC_SIZE)[:OUT_SIZE]


def kernel(index, rate, starttime, endtime, t):
    idx2 = index.reshape(ROWS, LANES)
    rt2 = rate.reshape(ROWS, LANES)
    st2 = starttime.reshape(ROWS, LANES)
    en2 = endtime.reshape(ROWS, LANES)
    t16 = jnp.broadcast_to(t.astype(jnp.float32), (16,))
    partials = _sc_call(idx2, rt2, st2, en2, t16)
    return _combine(partials)


# speedup vs baseline: 19.2142x; 1.6878x over previous
import jax
import jax.numpy as jnp
from jax import lax
from jax.experimental import pallas as pl
from jax.experimental.pallas import tpu as pltpu
from jax.experimental.pallas import tpu_sc as plsc

N = 2_000_000
OUT_SIZE = 1_000_000
LANES = 128
ROWS = N // LANES
CHUNK_ROWS = 16
FULL_CHUNKS = ROWS // CHUNK_ROWS
TAIL_ROWS = ROWS - FULL_CHUNKS * CHUNK_ROWS
NUM_TILES = 32
ACC_SIZE = 1 << 20
STRIPE = ACC_SIZE // 16
ZBUF = 8192
MAX_ITERS = 16


def _sc_body(idx_hbm, rate_hbm, st_hbm, en_hbm, t_hbm, out_hbm,
             t_v, idx0, val0, st0, en0, idx1, val1, st1, en1, zbuf,
             sem_in0, sem_in1, sem_sc0, sem_sc1, sem_z, acc):
    cid = lax.axis_index("c")
    sid = lax.axis_index("s")
    wid = sid * 2 + cid
    bufs = ((idx0, val0, st0, en0, sem_in0, sem_sc0),
            (idx1, val1, st1, en1, sem_in1, sem_sc1))

    def zero_body(i, carry):
        zbuf[pl.ds(i * 16, 16)] = jnp.zeros((16,), jnp.float32)
        return carry
    lax.fori_loop(0, ZBUF // 16, zero_body, 0)
    for z in range(STRIPE // ZBUF):
        pltpu.async_copy(zbuf, acc.at[pl.ds(sid * STRIPE + z * ZBUF, ZBUF)], sem_z)
    for z in range(STRIPE // ZBUF):
        pltpu.make_async_copy(zbuf, acc.at[pl.ds(sid * STRIPE + z * ZBUF, ZBUF)], sem_z).wait()

    pltpu.sync_copy(t_hbm, t_v)
    t = t_v[...]

    plsc.subcore_barrier()

    nch = (FULL_CHUNKS - 1 - wid) // NUM_TILES + 1

    def row0_of(jj):
        return (wid + jj * NUM_TILES) * CHUNK_ROWS

    def start_in(slot, jj):
        idxb, valb, stb, enb, s_in, _ = bufs[slot]
        r0 = row0_of(jj)
        pltpu.async_copy(idx_hbm.at[pl.ds(r0, CHUNK_ROWS)], idxb, s_in)
        pltpu.async_copy(rate_hbm.at[pl.ds(r0, CHUNK_ROWS)], valb, s_in)
        pltpu.async_copy(st_hbm.at[pl.ds(r0, CHUNK_ROWS)], stb, s_in)
        pltpu.async_copy(en_hbm.at[pl.ds(r0, CHUNK_ROWS)], enb, s_in)

    def wait_in(slot, jj):
        idxb, valb, stb, enb, s_in, _ = bufs[slot]
        r0 = row0_of(jj)
        pltpu.make_async_copy(idx_hbm.at[pl.ds(r0, CHUNK_ROWS)], idxb, s_in).wait()
        pltpu.make_async_copy(rate_hbm.at[pl.ds(r0, CHUNK_ROWS)], valb, s_in).wait()
        pltpu.make_async_copy(st_hbm.at[pl.ds(r0, CHUNK_ROWS)], stb, s_in).wait()
        pltpu.make_async_copy(en_hbm.at[pl.ds(r0, CHUNK_ROWS)], enb, s_in).wait()

    def compute(slot, nrows):
        idxb, valb, stb, enb, _, _ = bufs[slot]
        for j in range(nrows):
            def mask_body(i, carry, j=j):
                c = i * 16
                s = stb[j, pl.ds(c, 16)]
                e = enb[j, pl.ds(c, 16)]
                r = valb[j, pl.ds(c, 16)]
                keep = (s <= t) & (t < e)
                valb[j, pl.ds(c, 16)] = jnp.where(keep, r, jnp.float32(0.0))
                return carry
            lax.fori_loop(0, LANES // 16, mask_body, 0)

    def fire_sc(slot):
        idxb, valb, _, _, _, s_sc = bufs[slot]
        for j in range(CHUNK_ROWS):
            pltpu.async_copy(valb.at[j], acc.at[idxb.at[j]], s_sc, add=True)

    def drain_sc(slot):
        idxb, valb, _, _, _, s_sc = bufs[slot]
        for j in range(CHUNK_ROWS):
            pltpu.make_async_copy(valb.at[j], acc.at[idxb.at[j]], s_sc).wait()

    start_in(0, 0)

    def iter_body(jj2, carry):
        for b in (0, 1):
            jj = jj2 * 2 + b
            o = 1 - b

            @pl.when(jj < nch)
            def _proc(jj=jj, b=b):
                wait_in(b, jj)
                compute(b, CHUNK_ROWS)
                fire_sc(b)

            @pl.when((jj >= 1) & (jj <= nch))
            def _drain(o=o):
                drain_sc(o)

            @pl.when(jj + 1 < nch)
            def _pre(jj=jj, o=o):
                start_in(o, jj + 1)
        return carry
    lax.fori_loop(0, MAX_ITERS, iter_body, 0)

    @pl.when(wid == 16)
    def _tail():
        idxb, valb, stb, enb, _, _ = bufs[0]
        r0 = FULL_CHUNKS * CHUNK_ROWS
        pltpu.sync_copy(idx_hbm.at[pl.ds(r0, TAIL_ROWS)], idxb.at[pl.ds(0, TAIL_ROWS)])
        pltpu.sync_copy(rate_hbm.at[pl.ds(r0, TAIL_ROWS)], valb.at[pl.ds(0, TAIL_ROWS)])
        pltpu.sync_copy(st_hbm.at[pl.ds(r0, TAIL_ROWS)], stb.at[pl.ds(0, TAIL_ROWS)])
        pltpu.sync_copy(en_hbm.at[pl.ds(r0, TAIL_ROWS)], enb.at[pl.ds(0, TAIL_ROWS)])
        compute(0, TAIL_ROWS)
        for j in range(TAIL_ROWS):
            pltpu.sync_copy(valb.at[j], acc.at[idxb.at[j]], add=True)

    plsc.subcore_barrier()

    pltpu.sync_copy(acc.at[pl.ds(sid * STRIPE, STRIPE)],
                    out_hbm.at[cid, pl.ds(sid * STRIPE, STRIPE)])


_sc_call = pl.kernel(
    _sc_body,
    mesh=plsc.VectorSubcoreMesh(core_axis_name="c", subcore_axis_name="s"),
    out_type=jax.ShapeDtypeStruct((2, ACC_SIZE), jnp.float32),
    scratch_types=[
        pltpu.VMEM((16,), jnp.float32),
        pltpu.VMEM((CHUNK_ROWS, LANES), jnp.int32),
        pltpu.VMEM((CHUNK_ROWS, LANES), jnp.float32),
        pltpu.VMEM((CHUNK_ROWS, LANES), jnp.float32),
        pltpu.VMEM((CHUNK_ROWS, LANES), jnp.float32),
        pltpu.VMEM((CHUNK_ROWS, LANES), jnp.int32),
        pltpu.VMEM((CHUNK_ROWS, LANES), jnp.float32),
        pltpu.VMEM((CHUNK_ROWS, LANES), jnp.float32),
        pltpu.VMEM((CHUNK_ROWS, LANES), jnp.float32),
        pltpu.VMEM((ZBUF,), jnp.float32),
        pltpu.SemaphoreType.DMA,
        pltpu.SemaphoreType.DMA,
        pltpu.SemaphoreType.DMA,
        pltpu.SemaphoreType.DMA,
        pltpu.SemaphoreType.DMA,
        pltpu.VMEM_SHARED((ACC_SIZE,), jnp.float32),
    ],
)


def _combine_body(p_ref, o_ref):
    o_ref[...] = p_ref[0] + p_ref[1]


def _combine(partials):
    p3 = partials.reshape(2, 1024, 1024)
    out = pl.pallas_call(
        _combine_body,
        out_shape=jax.ShapeDtypeStruct((1024, 1024), jnp.float32),
    )(p3)
    return out.reshape(ACC_SIZE)[:OUT_SIZE]


def kernel(index, rate, starttime, endtime, t):
    idx2 = index.reshape(ROWS, LANES)
    rt2 = rate.reshape(ROWS, LANES)
    st2 = starttime.reshape(ROWS, LANES)
    en2 = endtime.reshape(ROWS, LANES)
    t16 = jnp.broadcast_to(t.astype(jnp.float32), (16,))
    partials = _sc_call(idx2, rt2, st2, en2, t16)
    return _combine(partials)


# speedup vs baseline: 23.1063x; 1.2026x over previous
import jax
import jax.numpy as jnp
from jax import lax
from jax.experimental import pallas as pl
from jax.experimental.pallas import tpu as pltpu
from jax.experimental.pallas import tpu_sc as plsc

N = 2_000_000
OUT_SIZE = 1_000_000
CHUNK = 2048
FULL_CHUNKS = N // CHUNK
TAIL = N - FULL_CHUNKS * CHUNK
NUM_TILES = 32
ACC_SIZE = 1 << 20
STRIPE = ACC_SIZE // 16
ZBUF = 8192
MAX_ITERS = 16


def _sc_body(idx_hbm, rate_hbm, st_hbm, en_hbm, t_hbm, out_hbm,
             t_v, idx0, rt0, st0, en0, idx1, rt1, st1, en1,
             zbuf, sem_in0, sem_in1, sem_sc0, sem_sc1, sem_z, acc):
    cid = lax.axis_index("c")
    sid = lax.axis_index("s")
    wid = sid * 2 + cid
    bufs = ((idx0, rt0, st0, en0, sem_in0, sem_sc0),
            (idx1, rt1, st1, en1, sem_in1, sem_sc1))

    def zero_body(i, carry):
        zbuf[pl.ds(i * 16, 16)] = jnp.zeros((16,), jnp.float32)
        return carry
    lax.fori_loop(0, ZBUF // 16, zero_body, 0)
    for z in range(STRIPE // ZBUF):
        pltpu.async_copy(zbuf, acc.at[pl.ds(sid * STRIPE + z * ZBUF, ZBUF)], sem_z)
    for z in range(STRIPE // ZBUF):
        pltpu.make_async_copy(zbuf, acc.at[pl.ds(sid * STRIPE + z * ZBUF, ZBUF)], sem_z).wait()

    pltpu.sync_copy(t_hbm, t_v)
    t = t_v[...]

    plsc.subcore_barrier()

    nch = (FULL_CHUNKS - 1 - wid) // NUM_TILES + 1

    def base_of(jj):
        return (wid + jj * NUM_TILES) * CHUNK

    def start_in(slot, jj):
        idxb, rtb, stb, enb, s_in, _ = bufs[slot]
        e0 = base_of(jj)
        pltpu.async_copy(idx_hbm.at[pl.ds(e0, CHUNK)], idxb, s_in)
        pltpu.async_copy(rate_hbm.at[pl.ds(e0, CHUNK)], rtb, s_in)
        pltpu.async_copy(st_hbm.at[pl.ds(e0, CHUNK)], stb, s_in)
        pltpu.async_copy(en_hbm.at[pl.ds(e0, CHUNK)], enb, s_in)

    def wait_in(slot, jj):
        idxb, rtb, stb, enb, s_in, _ = bufs[slot]
        e0 = base_of(jj)
        pltpu.make_async_copy(idx_hbm.at[pl.ds(e0, CHUNK)], idxb, s_in).wait()
        pltpu.make_async_copy(rate_hbm.at[pl.ds(e0, CHUNK)], rtb, s_in).wait()
        pltpu.make_async_copy(st_hbm.at[pl.ds(e0, CHUNK)], stb, s_in).wait()
        pltpu.make_async_copy(en_hbm.at[pl.ds(e0, CHUNK)], enb, s_in).wait()

    def mask_rates(slot, nevents):
        idxb, rtb, stb, enb, _, _ = bufs[slot]

        def grp(i, carry):
            c = i * 16
            s = stb[pl.ds(c, 16)]
            e = enb[pl.ds(c, 16)]
            r = rtb[pl.ds(c, 16)]
            m = (s <= t) & (t < e)
            rtb[pl.ds(c, 16)] = jnp.where(m, r, jnp.float32(0.0))
            return carry
        lax.fori_loop(0, nevents // 16, grp, 0)

    def fire_sc(slot, nrows):
        idxb, rtb, _, _, _, s_sc = bufs[slot]

        def f(j, carry):
            pltpu.async_copy(rtb.at[pl.ds(j * 128, 128)],
                             acc.at[idxb.at[pl.ds(j * 128, 128)]], s_sc, add=True)
            return carry
        lax.fori_loop(0, nrows, f, 0)

    def drain_sc(slot, nrows):
        idxb, rtb, _, _, _, s_sc = bufs[slot]

        def f(j, carry):
            pltpu.make_async_copy(rtb.at[pl.ds(j * 128, 128)],
                                  acc.at[idxb.at[pl.ds(j * 128, 128)]], s_sc).wait()
            return carry
        lax.fori_loop(0, nrows, f, 0)

    start_in(0, 0)

    def iter_body(jj2, carry):
        for b in (0, 1):
            jj = jj2 * 2 + b
            o = 1 - b

            @pl.when(jj < nch)
            def _proc(jj=jj, b=b):
                wait_in(b, jj)
                mask_rates(b, CHUNK)
                fire_sc(b, CHUNK // 128)

            @pl.when((jj >= 1) & (jj <= nch))
            def _drain(o=o):
                drain_sc(o, CHUNK // 128)

            @pl.when(jj + 1 < nch)
            def _pre(jj=jj, o=o):
                start_in(o, jj + 1)
        return carry
    lax.fori_loop(0, MAX_ITERS, iter_body, 0)

    @pl.when(wid == 16)
    def _tail():
        idxb, rtb, stb, enb, _, _ = bufs[0]
        e0 = FULL_CHUNKS * CHUNK
        pltpu.sync_copy(idx_hbm.at[pl.ds(e0, TAIL)], idxb.at[pl.ds(0, TAIL)])
        pltpu.sync_copy(rate_hbm.at[pl.ds(e0, TAIL)], rtb.at[pl.ds(0, TAIL)])
        pltpu.sync_copy(st_hbm.at[pl.ds(e0, TAIL)], stb.at[pl.ds(0, TAIL)])
        pltpu.sync_copy(en_hbm.at[pl.ds(e0, TAIL)], enb.at[pl.ds(0, TAIL)])
        mask_rates(0, TAIL)

        def f(j, carry):
            pltpu.sync_copy(rtb.at[pl.ds(j * 128, 128)],
                            acc.at[idxb.at[pl.ds(j * 128, 128)]], add=True)
            return carry
        lax.fori_loop(0, TAIL // 128, f, 0)

    plsc.subcore_barrier()

    pltpu.sync_copy(acc.at[pl.ds(sid * STRIPE, STRIPE)],
                    out_hbm.at[cid, pl.ds(sid * STRIPE, STRIPE)])


_sc_call = pl.kernel(
    _sc_body,
    mesh=plsc.VectorSubcoreMesh(core_axis_name="c", subcore_axis_name="s"),
    out_type=jax.ShapeDtypeStruct((2, ACC_SIZE), jnp.float32),
    scratch_types=[
        pltpu.VMEM((16,), jnp.float32),
        pltpu.VMEM((CHUNK,), jnp.int32),
        pltpu.VMEM((CHUNK,), jnp.float32),
        pltpu.VMEM((CHUNK,), jnp.float32),
        pltpu.VMEM((CHUNK,), jnp.float32),
        pltpu.VMEM((CHUNK,), jnp.int32),
        pltpu.VMEM((CHUNK,), jnp.float32),
        pltpu.VMEM((CHUNK,), jnp.float32),
        pltpu.VMEM((CHUNK,), jnp.float32),
        pltpu.VMEM((ZBUF,), jnp.float32),
        pltpu.SemaphoreType.DMA,
        pltpu.SemaphoreType.DMA,
        pltpu.SemaphoreType.DMA,
        pltpu.SemaphoreType.DMA,
        pltpu.SemaphoreType.DMA,
        pltpu.VMEM_SHARED((ACC_SIZE,), jnp.float32),
    ],
)


def _combine_body(p_ref, o_ref):
    o_ref[...] = p_ref[0] + p_ref[1]


def _combine(partials):
    p3 = partials.reshape(2, 1024, 1024)
    out = pl.pallas_call(
        _combine_body,
        out_shape=jax.ShapeDtypeStruct((1024, 1024), jnp.float32),
    )(p3)
    return out.reshape(ACC_SIZE)[:OUT_SIZE]


def kernel(index, rate, starttime, endtime, t):
    t16 = jnp.broadcast_to(t.astype(jnp.float32), (16,))
    partials = _sc_call(index, rate, starttime, endtime, t16)
    return _combine(partials)


# speedup vs baseline: 28.3653x; 1.2276x over previous
import jax
import jax.numpy as jnp
from jax import lax
from jax.experimental import pallas as pl
from jax.experimental.pallas import tpu as pltpu
from jax.experimental.pallas import tpu_sc as plsc

N = 2_000_000
OUT_SIZE = 1_000_000
ROWS = N // 128
MROWS = ROWS // 5
CHUNK = 8192
FULL_CHUNKS = N // CHUNK
TAIL = N - FULL_CHUNKS * CHUNK
NUM_TILES = 32
ACC_SIZE = 1 << 20
STRIPE = ACC_SIZE // 16
ZBUF = 8192
MAX_ITERS = 5


def _mask_body(t_ref, rt_ref, st_ref, en_ref, o_ref):
    t = t_ref[0, 0]
    m = (st_ref[...] <= t) & (t < en_ref[...])
    o_ref[...] = jnp.where(m, rt_ref[...], jnp.float32(0.0))


def _masked_rate(t2d, rate2d, st2d, en2d):
    return pl.pallas_call(
        _mask_body,
        out_shape=jax.ShapeDtypeStruct((ROWS, 128), jnp.float32),
    )(t2d, rate2d, st2d, en2d)


def _sc_body(idx_hbm, mr_hbm, out_hbm,
             idx0, mr0, idx1, mr1, zbuf,
             sem_in0, sem_in1, sem_sc0, sem_sc1, sem_z, acc):
    cid = lax.axis_index("c")
    sid = lax.axis_index("s")
    wid = sid * 2 + cid
    bufs = ((idx0, mr0, sem_in0, sem_sc0),
            (idx1, mr1, sem_in1, sem_sc1))

    def zero_body(i, carry):
        zbuf[pl.ds(i * 16, 16)] = jnp.zeros((16,), jnp.float32)
        return carry
    lax.fori_loop(0, ZBUF // 16, zero_body, 0)
    for z in range(STRIPE // ZBUF):
        pltpu.async_copy(zbuf, acc.at[pl.ds(sid * STRIPE + z * ZBUF, ZBUF)], sem_z)
    for z in range(STRIPE // ZBUF):
        pltpu.make_async_copy(zbuf, acc.at[pl.ds(sid * STRIPE + z * ZBUF, ZBUF)], sem_z).wait()

    plsc.subcore_barrier()

    nch = (FULL_CHUNKS - 1 - wid) // NUM_TILES + 1

    def base_of(jj):
        return (wid + jj * NUM_TILES) * CHUNK

    def start_in(slot, jj):
        idxb, mrb, s_in, _ = bufs[slot]
        e0 = base_of(jj)
        pltpu.async_copy(idx_hbm.at[pl.ds(e0, CHUNK)], idxb, s_in)
        pltpu.async_copy(mr_hbm.at[pl.ds(e0, CHUNK)], mrb, s_in)

    def wait_in(slot, jj):
        idxb, mrb, s_in, _ = bufs[slot]
        e0 = base_of(jj)
        pltpu.make_async_copy(idx_hbm.at[pl.ds(e0, CHUNK)], idxb, s_in).wait()
        pltpu.make_async_copy(mr_hbm.at[pl.ds(e0, CHUNK)], mrb, s_in).wait()

    def fire_sc(slot):
        idxb, mrb, _, s_sc = bufs[slot]
        pltpu.async_copy(mrb, acc.at[idxb], s_sc, add=True)

    def drain_sc(slot):
        idxb, mrb, _, s_sc = bufs[slot]
        pltpu.make_async_copy(mrb, acc.at[idxb], s_sc).wait()

    start_in(0, 0)

    def iter_body(jj2, carry):
        for b in (0, 1):
            jj = jj2 * 2 + b
            o = 1 - b

            @pl.when(jj < nch)
            def _proc(jj=jj, b=b):
                wait_in(b, jj)
                fire_sc(b)

            @pl.when((jj >= 1) & (jj <= nch))
            def _drain(o=o):
                drain_sc(o)

            @pl.when(jj + 1 < nch)
            def _pre(jj=jj, o=o):
                start_in(o, jj + 1)
        return carry
    lax.fori_loop(0, MAX_ITERS, iter_body, 0)

    @pl.when(wid == 16)
    def _tail():
        idxb, mrb, _, _ = bufs[0]
        e0 = FULL_CHUNKS * CHUNK
        pltpu.sync_copy(idx_hbm.at[pl.ds(e0, TAIL)], idxb.at[pl.ds(0, TAIL)])
        pltpu.sync_copy(mr_hbm.at[pl.ds(e0, TAIL)], mrb.at[pl.ds(0, TAIL)])
        pltpu.sync_copy(mrb.at[pl.ds(0, TAIL)],
                        acc.at[idxb.at[pl.ds(0, TAIL)]], add=True)

    plsc.subcore_barrier()

    pltpu.sync_copy(acc.at[pl.ds(sid * STRIPE, STRIPE)],
                    out_hbm.at[cid, pl.ds(sid * STRIPE, STRIPE)])


_sc_call = pl.kernel(
    _sc_body,
    mesh=plsc.VectorSubcoreMesh(core_axis_name="c", subcore_axis_name="s"),
    out_type=jax.ShapeDtypeStruct((2, ACC_SIZE), jnp.float32),
    scratch_types=[
        pltpu.VMEM((CHUNK,), jnp.int32),
        pltpu.VMEM((CHUNK,), jnp.float32),
        pltpu.VMEM((CHUNK,), jnp.int32),
        pltpu.VMEM((CHUNK,), jnp.float32),
        pltpu.VMEM((ZBUF,), jnp.float32),
        pltpu.SemaphoreType.DMA,
        pltpu.SemaphoreType.DMA,
        pltpu.SemaphoreType.DMA,
        pltpu.SemaphoreType.DMA,
        pltpu.SemaphoreType.DMA,
        pltpu.VMEM_SHARED((ACC_SIZE,), jnp.float32),
    ],
)


def _combine_body(p_ref, o_ref):
    o_ref[...] = p_ref[0] + p_ref[1]


def _combine(partials):
    p3 = partials.reshape(2, 1024, 1024)
    out = pl.pallas_call(
        _combine_body,
        out_shape=jax.ShapeDtypeStruct((1024, 1024), jnp.float32),
    )(p3)
    return out.reshape(ACC_SIZE)[:OUT_SIZE]


def kernel(index, rate, starttime, endtime, t):
    t2d = t.astype(jnp.float32).reshape(1, 1)
    mr = _masked_rate(t2d,
                      rate.reshape(ROWS, 128),
                      starttime.reshape(ROWS, 128),
                      endtime.reshape(ROWS, 128)).reshape(N)
    partials = _sc_call(index, mr)
    return _combine(partials)
